# baseline (speedup 1.0000x reference)
import functools
import math

import jax
import jax.numpy as jnp
from jax import lax
from jax.experimental import pallas as pl
from jax.experimental.pallas import tpu as pltpu
from jax.experimental.pallas import tpu_sc as plsc

D_MODEL = 128
N_I = 4096
N_J = 50
NUM_ROWS_OUT = N_I * N_J
_SCALE = math.sqrt(float(D_MODEL))

_NC = 2
_NS = 16
_NW = _NC * _NS
_IBLK = N_I // _NW
_NBUF = 5
_NGRP = N_J // _NBUF
_ROW_UNROLL = 4
_LANES_PER_ROW = D_MODEL // 16


@functools.partial(
    pl.kernel,
    out_type=jax.ShapeDtypeStruct((NUM_ROWS_OUT, D_MODEL), jnp.float32),
    mesh=plsc.VectorSubcoreMesh(core_axis_name="c", subcore_axis_name="s"),
    scratch_types=[
        pltpu.VMEM((N_J, _IBLK), jnp.int32),
        pltpu.VMEM((_IBLK, D_MODEL), jnp.float32),
        pltpu.VMEM((_IBLK, D_MODEL), jnp.float32),
        pltpu.VMEM((_IBLK, D_MODEL), jnp.float32),
        pltpu.VMEM((_IBLK, D_MODEL), jnp.float32),
        pltpu.VMEM((_IBLK, D_MODEL), jnp.float32),
        pltpu.SemaphoreType.DMA((_NBUF,)),
        pltpu.SemaphoreType.DMA((_NBUF,)),
    ],
)
def _gather_scale(xt_hbm, table_hbm, out_hbm, idx_v, b0, b1, b2, b3, b4,
                  gsem, ssem):
    wid = lax.axis_index("s") * _NC + lax.axis_index("c")
    i0 = wid * _IBLK
    pltpu.sync_copy(xt_hbm.at[:, pl.ds(i0, _IBLK)], idx_v)

    bufs = (b0, b1, b2, b3, b4)

    def start_gather(j, b):
        pltpu.make_async_copy(
            table_hbm.at[idx_v.at[j]], bufs[b], gsem.at[b]
        ).start()

    def wait_gather(b):
        pltpu.make_async_copy(
            table_hbm.at[idx_v.at[0]], bufs[b], gsem.at[b]
        ).wait()

    def start_scatter(j, b):
        pltpu.make_async_copy(
            bufs[b], out_hbm.at[pl.ds(j * N_I + i0, _IBLK)], ssem.at[b]
        ).start()

    def wait_scatter(b):
        pltpu.make_async_copy(
            bufs[b], out_hbm.at[pl.ds(i0, _IBLK)], ssem.at[b]
        ).wait()

    def scale_buf(buf):
        def row_body(r, _):
            r0 = r * _ROW_UNROLL
            for dj in range(_ROW_UNROLL):
                for k in range(_LANES_PER_ROW):
                    sl = (r0 + dj, pl.ds(k * 16, 16))
                    buf[sl] = buf[sl] * _SCALE
            return _

        lax.fori_loop(0, _IBLK // _ROW_UNROLL, row_body, None)

    def chunk_step(j, b, *, first=False, last=False):
        if not first:
            wait_scatter((b + 4) % _NBUF)
        if not last:
            start_gather(j + 4, (b + 4) % _NBUF)
        wait_gather(b)
        scale_buf(bufs[b])
        start_scatter(j, b)

    for b in range(_NBUF - 1):
        start_gather(b, b)
    chunk_step(0, 0, first=True)
    for b in range(1, _NBUF):
        chunk_step(b, b)

    @pl.loop(1, _NGRP - 1)
    def _middle(g):
        j0 = g * _NBUF
        for b in range(_NBUF):
            chunk_step(j0 + b, b)

    j0 = (_NGRP - 1) * _NBUF
    for b in range(_NBUF):
        chunk_step(j0 + b, b, last=(b >= 1))
    wait_scatter((N_J - 1) % _NBUF)


def kernel(x, table):
    xt = x.T.astype(jnp.int32)
    out_t = _gather_scale(xt, table)
    return out_t.reshape(N_J, N_I, D_MODEL).transpose(1, 0, 2)


# speedup vs baseline: 1.0221x; 1.0221x over previous
import functools
import math

import jax
import jax.numpy as jnp
from jax import lax
from jax.experimental import pallas as pl
from jax.experimental.pallas import tpu as pltpu
from jax.experimental.pallas import tpu_sc as plsc

D_MODEL = 128
N_I = 4096
N_J = 50
NUM_ROWS_OUT = N_I * N_J
_SCALE = math.sqrt(float(D_MODEL))

_NC = 2
_NS = 16
_NW = _NC * _NS
_IBLK = N_I // _NW
_NBUF = 3
_AHEAD = 2
_NGRP = 16
_ROW_UNROLL = 4
_LANES_PER_ROW = D_MODEL // 16


@functools.partial(
    pl.kernel,
    out_type=jax.ShapeDtypeStruct((NUM_ROWS_OUT, D_MODEL), jnp.float32),
    mesh=plsc.VectorSubcoreMesh(core_axis_name="c", subcore_axis_name="s"),
    scratch_types=[
        pltpu.VMEM((N_J, _IBLK), jnp.int32),
        pltpu.VMEM((_IBLK, D_MODEL), jnp.float32),
        pltpu.VMEM((_IBLK, D_MODEL), jnp.float32),
        pltpu.VMEM((_IBLK, D_MODEL), jnp.float32),
        pltpu.SemaphoreType.DMA((_NBUF,)),
        pltpu.SemaphoreType.DMA((_NBUF,)),
    ],
)
def _gather_scale(xt_hbm, table_hbm, out_hbm, idx_v, b0, b1, b2, gsem, ssem):
    wid = lax.axis_index("s") * _NC + lax.axis_index("c")
    i0 = wid * _IBLK
    pltpu.sync_copy(xt_hbm.at[:, pl.ds(i0, _IBLK)], idx_v)

    bufs = (b0, b1, b2)

    def start_gather(j, b):
        pltpu.make_async_copy(
            table_hbm.at[idx_v.at[j]], bufs[b], gsem.at[b]
        ).start()

    def wait_gather(b):
        pltpu.make_async_copy(
            table_hbm.at[idx_v.at[0]], bufs[b], gsem.at[b]
        ).wait()

    def start_scatter(j, b):
        pltpu.make_async_copy(
            bufs[b], out_hbm.at[pl.ds(j * N_I + i0, _IBLK)], ssem.at[b]
        ).start()

    def wait_scatter(b):
        pltpu.make_async_copy(
            bufs[b], out_hbm.at[pl.ds(i0, _IBLK)], ssem.at[b]
        ).wait()

    def scale_buf(buf):
        def row_body(r, _):
            r0 = r * _ROW_UNROLL
            for dj in range(_ROW_UNROLL):
                for k in range(_LANES_PER_ROW):
                    sl = (r0 + dj, pl.ds(k * 16, 16))
                    buf[sl] = buf[sl] * _SCALE
            return _

        lax.fori_loop(0, _IBLK // _ROW_UNROLL, row_body, None)

    def chunk_step(j, b, *, first=False, last=False):
        if not first:
            wait_scatter((b + _NBUF - 1) % _NBUF)
        if not last:
            start_gather(j + _AHEAD, (b + _AHEAD) % _NBUF)
        wait_gather(b)
        scale_buf(bufs[b])
        start_scatter(j, b)

    for b in range(_AHEAD):
        start_gather(b, b)
    chunk_step(0, 0, first=True)
    for b in range(1, _NBUF):
        chunk_step(b, b)

    @pl.loop(1, _NGRP)
    def _middle(g):
        j0 = g * _NBUF
        for b in range(_NBUF):
            chunk_step(j0 + b, b)

    for j in range(_NGRP * _NBUF, N_J):
        chunk_step(j, j % _NBUF, last=True)
    wait_scatter((N_J - 1) % _NBUF)


def kernel(x, table):
    xt = x.T.astype(jnp.int32)
    out_t = _gather_scale(xt, table)
    return out_t.reshape(N_J, N_I, D_MODEL).transpose(1, 0, 2)
